# baseline (speedup 1.0000x reference)
import functools

import jax
import jax.numpy as jnp
from jax import lax
from jax.experimental import pallas as pl
from jax.experimental.pallas import tpu as pltpu

E = 64
TOPK = 2
TB = 256


def _router_body(x_ref, rw_ref, idx_ref, prob_ref):
    logits = lax.dot_general(x_ref[...], rw_ref[...],
                             (((1,), (1,)), ((), ())),
                             preferred_element_type=jnp.float32)
    m1 = jnp.max(logits, axis=1)
    a1 = jnp.argmax(logits, axis=1).astype(jnp.int32)
    cols = lax.broadcasted_iota(jnp.int32, logits.shape, 1)
    masked = jnp.where(cols == a1[:, None], -jnp.inf, logits)
    m2 = jnp.max(masked, axis=1)
    a2 = jnp.argmax(masked, axis=1).astype(jnp.int32)
    e2 = jnp.exp(m2 - m1)
    p1 = 1.0 / (1.0 + e2)
    p2 = e2 / (1.0 + e2)
    idx_ref[...] = jnp.stack([a1, a2], axis=1)
    prob_ref[...] = jnp.stack([p1, p2], axis=1)


def _router(x_flat, router_w):
    n, d = x_flat.shape
    grid = n // TB
    return pl.pallas_call(
        _router_body,
        grid=(grid,),
        in_specs=[
            pl.BlockSpec((TB, d), lambda i: (i, 0)),
            pl.BlockSpec((E, d), lambda i: (0, 0)),
        ],
        out_specs=[
            pl.BlockSpec((TB, TOPK), lambda i: (i, 0)),
            pl.BlockSpec((TB, TOPK), lambda i: (i, 0)),
        ],
        out_shape=[
            jax.ShapeDtypeStruct((n, TOPK), jnp.int32),
            jax.ShapeDtypeStruct((n, TOPK), jnp.float32),
        ],
    )(x_flat, router_w)


def _gelu_exact(v):
    return 0.5 * v * (1.0 + lax.erf(v * 0.7071067811865476))


def _expert_body(idx_ref, prob_ref, scale_ref, x_ref, a_ref, b_ref, bias_ref,
                 out_ref, *, do_gelu):
    def tok(n, _):
        e1 = idx_ref[n, 0]
        e2 = idx_ref[n, 1]
        p1 = prob_ref[n, 0]
        p2 = prob_ref[n, 1]
        xn = x_ref[n]
        t1 = jnp.dot(a_ref[e1], xn, preferred_element_type=jnp.float32)
        y1 = lax.dot_general(t1, b_ref[e1], (((1,), (1,)), ((), ())),
                             preferred_element_type=jnp.float32)
        t2 = jnp.dot(a_ref[e2], xn, preferred_element_type=jnp.float32)
        y2 = lax.dot_general(t2, b_ref[e2], (((1,), (1,)), ((), ())),
                             preferred_element_type=jnp.float32)
        out_ref[n] = p1 * y1 + p2 * y2
        return 0

    lax.fori_loop(0, TB, tok, 0)
    v = out_ref[...] * scale_ref[0] + bias_ref[...]
    if do_gelu:
        v = _gelu_exact(v)
    out_ref[...] = v


def _expert_layer(x3, idx, prob, A, B, bias2d, scale, do_gelu):
    n, d1, d2 = x3.shape
    _, o1, _ = A.shape
    _, o2, _ = B.shape
    grid = n // TB
    body = functools.partial(_expert_body, do_gelu=do_gelu)
    return pl.pallas_call(
        body,
        grid=(grid,),
        in_specs=[
            pl.BlockSpec((TB, TOPK), lambda i: (i, 0), memory_space=pltpu.SMEM),
            pl.BlockSpec((TB, TOPK), lambda i: (i, 0), memory_space=pltpu.SMEM),
            pl.BlockSpec((1,), lambda i: (0,), memory_space=pltpu.SMEM),
            pl.BlockSpec((TB, d1, d2), lambda i: (i, 0, 0)),
            pl.BlockSpec(A.shape, lambda i: (0, 0, 0)),
            pl.BlockSpec(B.shape, lambda i: (0, 0, 0)),
            pl.BlockSpec((1, o1, o2), lambda i: (0, 0, 0)),
        ],
        out_specs=pl.BlockSpec((TB, o1, o2), lambda i: (i, 0, 0)),
        out_shape=jax.ShapeDtypeStruct((n, o1, o2), jnp.float32),
    )(idx, prob, scale, x3, A, B, bias2d)


def _kmoe_layer(x_flat, router_w, A, B, scale, bias, d1, d2, o1, o2, do_gelu):
    n = x_flat.shape[0]
    idx, prob = _router(x_flat, router_w)
    x3 = x_flat.reshape(n, d1, d2)
    bias2d = bias.reshape(1, o1, o2)
    out3 = _expert_layer(x3, idx, prob, A, B, bias2d, scale, do_gelu)
    return out3.reshape(n, o1 * o2)


def kernel(x, router_up, A_up, B_up, scale_up, bias_up,
           router_down, A_down, B_down, scale_down, bias_down):
    orig_shape = x.shape
    n = x.shape[0] * x.shape[1]
    x_flat = x.reshape(n, x.shape[2])
    h = _kmoe_layer(x_flat, router_up, A_up, B_up, scale_up, bias_up,
                    32, 32, 64, 64, do_gelu=True)
    y = _kmoe_layer(h, router_down, A_down, B_down, scale_down, bias_down,
                    64, 64, 32, 32, do_gelu=False)
    return y.reshape(orig_shape[:-1] + (32 * 32,))


# speedup vs baseline: 3.0583x; 3.0583x over previous
import functools

import jax
import jax.numpy as jnp
from jax import lax
from jax.experimental import pallas as pl
from jax.experimental.pallas import tpu as pltpu

E = 64
TOPK = 2
TB = 256
UNROLL = 8


def _router_body(x_ref, rw_ref, idx_ref, prob_ref):
    logits = lax.dot_general(x_ref[...], rw_ref[...],
                             (((1,), (1,)), ((), ())),
                             preferred_element_type=jnp.float32)
    m1 = jnp.max(logits, axis=1)
    a1 = jnp.argmax(logits, axis=1).astype(jnp.int32)
    cols = lax.broadcasted_iota(jnp.int32, logits.shape, 1)
    masked = jnp.where(cols == a1[:, None], -jnp.inf, logits)
    m2 = jnp.max(masked, axis=1)
    a2 = jnp.argmax(masked, axis=1).astype(jnp.int32)
    e2 = jnp.exp(m2 - m1)
    p1 = 1.0 / (1.0 + e2)
    p2 = e2 / (1.0 + e2)
    idx_ref[...] = jnp.stack([a1, a2], axis=1)
    prob_ref[...] = jnp.stack([p1, p2], axis=1)


def _router(x_flat, router_w):
    n, d = x_flat.shape
    grid = n // TB
    return pl.pallas_call(
        _router_body,
        grid=(grid,),
        in_specs=[
            pl.BlockSpec((TB, d), lambda i: (i, 0)),
            pl.BlockSpec((E, d), lambda i: (0, 0)),
        ],
        out_specs=[
            pl.BlockSpec((TB, TOPK), lambda i: (i, 0)),
            pl.BlockSpec((TB, TOPK), lambda i: (i, 0)),
        ],
        out_shape=[
            jax.ShapeDtypeStruct((n, TOPK), jnp.int32),
            jax.ShapeDtypeStruct((n, TOPK), jnp.float32),
        ],
    )(x_flat, router_w)


def _gelu_exact(v):
    return 0.5 * v * (1.0 + lax.erf(v * 0.7071067811865476))


def _expert_body(idx_ref, prob_ref, scale_ref, x_ref, a_ref, b_ref, bias_ref,
                 out_ref, *, do_gelu):
    o1 = out_ref.shape[1]

    def tok(n, _):
        for u in range(UNROLL):
            m = n * UNROLL + u
            e1 = idx_ref[m, 0]
            e2 = idx_ref[m, 1]
            p1 = prob_ref[m, 0]
            p2 = prob_ref[m, 1]
            xn = x_ref[m]
            a_cat = jnp.concatenate([a_ref[e1], a_ref[e2]], axis=0)
            t12 = jnp.dot(a_cat, xn, preferred_element_type=jnp.float32)
            t_cat = jnp.concatenate([p1 * t12[:o1], p2 * t12[o1:]], axis=1)
            b_cat = jnp.concatenate([b_ref[e1], b_ref[e2]], axis=1)
            out_ref[m] = lax.dot_general(
                t_cat, b_cat, (((1,), (1,)), ((), ())),
                preferred_element_type=jnp.float32)
        return 0

    lax.fori_loop(0, TB // UNROLL, tok, 0)
    v = out_ref[...] * scale_ref[0] + bias_ref[...]
    if do_gelu:
        v = _gelu_exact(v)
    out_ref[...] = v


def _expert_layer(x3, idx, prob, A, B, bias2d, scale, do_gelu):
    n, d1, d2 = x3.shape
    _, o1, _ = A.shape
    _, o2, _ = B.shape
    grid = n // TB
    body = functools.partial(_expert_body, do_gelu=do_gelu)
    return pl.pallas_call(
        body,
        grid=(grid,),
        in_specs=[
            pl.BlockSpec((TB, TOPK), lambda i: (i, 0), memory_space=pltpu.SMEM),
            pl.BlockSpec((TB, TOPK), lambda i: (i, 0), memory_space=pltpu.SMEM),
            pl.BlockSpec((1,), lambda i: (0,), memory_space=pltpu.SMEM),
            pl.BlockSpec((TB, d1, d2), lambda i: (i, 0, 0)),
            pl.BlockSpec(A.shape, lambda i: (0, 0, 0)),
            pl.BlockSpec(B.shape, lambda i: (0, 0, 0)),
            pl.BlockSpec((1, o1, o2), lambda i: (0, 0, 0)),
        ],
        out_specs=pl.BlockSpec((TB, o1, o2), lambda i: (i, 0, 0)),
        out_shape=jax.ShapeDtypeStruct((n, o1, o2), jnp.float32),
    )(idx, prob, scale, x3, A, B, bias2d)


def _kmoe_layer(x_flat, router_w, A, B, scale, bias, d1, d2, o1, o2, do_gelu):
    n = x_flat.shape[0]
    idx, prob = _router(x_flat, router_w)
    x3 = x_flat.reshape(n, d1, d2)
    bias2d = bias.reshape(1, o1, o2)
    out3 = _expert_layer(x3, idx, prob, A, B, bias2d, scale, do_gelu)
    return out3.reshape(n, o1 * o2)


def kernel(x, router_up, A_up, B_up, scale_up, bias_up,
           router_down, A_down, B_down, scale_down, bias_down):
    orig_shape = x.shape
    n = x.shape[0] * x.shape[1]
    x_flat = x.reshape(n, x.shape[2])
    h = _kmoe_layer(x_flat, router_up, A_up, B_up, scale_up, bias_up,
                    32, 32, 64, 64, do_gelu=True)
    y = _kmoe_layer(h, router_down, A_down, B_down, scale_down, bias_down,
                    64, 64, 32, 32, do_gelu=False)
    return y.reshape(orig_shape[:-1] + (32 * 32,))
